# baseline (speedup 1.0000x reference)
import functools

import numpy as np
import jax
import jax.numpy as jnp
from jax import lax
from jax.experimental import pallas as pl
from jax.experimental.pallas import tpu as pltpu
from jax.experimental.pallas import tpu_sc as plsc

_COS2 = float(np.cos(np.deg2rad(20.0)) ** 2)
_LANES = 128
_SC_L = 16
_CH = 64


def _tc_body(scal_ref, x_ref, y_ref, v_ref, p_ref, o_ref, vx_ref, vy_ref,
             *, n_veh: int, n_pad: int):
    X = x_ref[...]
    Y = y_ref[...]
    V = v_ref[...]
    P = p_ref[...]
    C = jnp.cos(P)
    S = jnp.sin(P)
    VX = V * C
    VY = V * S
    vx_ref[...] = VX
    vy_ref[...] = VY
    inf = jnp.float32(np.inf)
    cnd0 = jnp.full((n_pad, _LANES), inf, jnp.float32)
    z = jnp.zeros((n_pad, _LANES), jnp.float32)

    def body(a, carry):
        cnd, lvx, lvy = carry
        xa = x_ref[pl.ds(a, 1), :]
        ya = y_ref[pl.ds(a, 1), :]
        vxa = vx_ref[pl.ds(a, 1), :]
        vya = vy_ref[pl.ds(a, 1), :]
        dx = xa - X
        dy = ya - Y
        nd = dx * C + dy * S
        dr2 = dx * dx + dy * dy
        cone = nd * jnp.abs(nd) > dr2 * _COS2
        upd = cone & (nd < cnd)
        cnd = jnp.where(upd, nd, cnd)
        lvx = jnp.where(upd, vxa, lvx)
        lvy = jnp.where(upd, vya, lvy)
        return cnd, lvx, lvy

    cnd, lvx, lvy = lax.fori_loop(0, n_veh, body, (cnd0, z, z))

    L = scal_ref[0]
    v0 = scal_ref[1]
    s0 = scal_ref[2]
    dth = scal_ref[3]
    amax = scal_ref[4]
    bb = scal_ref[5]
    inv2 = 0.5 * lax.rsqrt(amax * bb)
    dvx = lvx - VX
    dvy = lvy - VY
    ndv = dvx * C + dvy * S
    sstar = s0 + V * dth + V * ndv * inv2
    sal = cnd - L
    t = V * (1.0 / v0)
    t2 = t * t
    af = amax * (1.0 - t2 * t2)
    r = sstar / sal
    act = af - amax * (r * r)
    o_ref[...] = jnp.where(sal == inf, af, act)


def _tc_idm(state, scal, n, n_pad):
    B = state.shape[0]
    st = state.reshape(B, -1, 5)
    pad = ((0, n_pad - n), (0, 0))
    xt = jnp.pad(st[..., 0].T, pad)
    yt = jnp.pad(st[..., 1].T, pad)
    vt = jnp.pad(st[..., 2].T, pad)
    pt = jnp.pad(st[..., 3].T, pad)
    grid = (B // _LANES,)
    body = functools.partial(_tc_body, n_veh=n, n_pad=n_pad)
    vspec = pl.BlockSpec((n_pad, _LANES), lambda i: (0, i))
    out = pl.pallas_call(
        body,
        grid=grid,
        in_specs=[
            pl.BlockSpec(memory_space=pltpu.SMEM),
            vspec, vspec, vspec, vspec,
        ],
        out_specs=vspec,
        out_shape=jax.ShapeDtypeStruct((n_pad, B), jnp.float32),
        scratch_shapes=[
            pltpu.VMEM((n_pad, _LANES), jnp.float32),
            pltpu.VMEM((n_pad, _LANES), jnp.float32),
        ],
        compiler_params=pltpu.CompilerParams(
            dimension_semantics=("parallel",)),
    )(scal, xt, yt, vt, pt)
    return out[:n].T


def _prep_body(scal_ref, v_ref, p_ref,
               c_ref, s_ref, vx_ref, vy_ref, ap_ref, bp_ref, af_ref):
    v = v_ref[...]
    p = p_ref[...]
    c = jnp.cos(p)
    s = jnp.sin(p)
    v0 = scal_ref[1]
    s0 = scal_ref[2]
    dth = scal_ref[3]
    amax = scal_ref[4]
    bb = scal_ref[5]
    c_ref[...] = c
    s_ref[...] = s
    vx_ref[...] = v * c
    vy_ref[...] = v * s
    sqa = jnp.sqrt(amax)
    ap_ref[...] = sqa * (s0 + v * dth)
    bp_ref[...] = (0.5 * lax.rsqrt(bb)) * v
    t = v * (1.0 / v0)
    t2 = t * t
    af_ref[...] = amax * (1.0 - t2 * t2)


def _tc_prep(x, y, v, p, scal):
    B, npad = v.shape
    blk = min(512, B)
    vspec = pl.BlockSpec((blk, npad), lambda i: (i, 0))
    shp = jax.ShapeDtypeStruct((B, npad), jnp.float32)
    return pl.pallas_call(
        _prep_body,
        grid=(B // blk,),
        in_specs=[pl.BlockSpec(memory_space=pltpu.SMEM), vspec, vspec],
        out_specs=[vspec] * 7,
        out_shape=[shp] * 7,
        compiler_params=pltpu.CompilerParams(
            dimension_semantics=("parallel",)),
    )(scal, v, p)


def _sc_idm_call(x, y, c, s, vx, vy, ap, bp, af, l16, n_veh, npad):
    B = x.shape[0] // npad
    ng = npad // _SC_L
    n_workers = 32
    per_w = B // n_workers
    chw = _CH * npad
    mesh = plsc.VectorSubcoreMesh(core_axis_name="c", subcore_axis_name="s")

    @functools.partial(
        pl.kernel, mesh=mesh,
        out_type=jax.ShapeDtypeStruct((B * npad,), jnp.float32),
        scratch_types=[pltpu.VMEM((chw,), jnp.float32)] * 10
                      + [pltpu.VMEM((_SC_L,), jnp.float32)],
        compiler_params=pltpu.CompilerParams(needs_layout_passes=False),
    )
    def sck(x_h, y_h, c_h, s_h, vx_h, vy_h, ap_h, bp_h, af_h, l_h, out_h,
            xv, yv, cv, sv, vxv, vyv, apv, bpv, afv, ov, lv):
        wid = lax.axis_index("s") * 2 + lax.axis_index("c")
        base0 = wid * (per_w * npad)
        pltpu.sync_copy(l_h, lv)
        L16 = lv[...]
        inf16 = jnp.full((_SC_L,), np.inf, jnp.float32)
        z16 = jnp.zeros((_SC_L,), jnp.float32)
        zi = jnp.zeros((_SC_L,), jnp.int32)

        for ci in range(per_w // _CH):
            cb = base0 + ci * chw
            for src, dst in ((x_h, xv), (y_h, yv), (c_h, cv), (s_h, sv),
                             (vx_h, vxv), (vy_h, vyv), (ap_h, apv),
                             (bp_h, bpv), (af_h, afv)):
                pltpu.sync_copy(src.at[pl.ds(cb, chw)], dst)

            def batch_body(bi, _):
                off = bi * npad

                def cand(a, carry):
                    ia = zi + (off + a)
                    xa = plsc.load_gather(xv, [ia])
                    ya = plsc.load_gather(yv, [ia])
                    vxa = plsc.load_gather(vxv, [ia])
                    vya = plsc.load_gather(vyv, [ia])
                    nc = []
                    for g in range(ng):
                        cnd, lx, ly = carry[3 * g:3 * g + 3]
                        sl = pl.ds(off + g * _SC_L, _SC_L)
                        dx = xa - xv[sl]
                        dy = ya - yv[sl]
                        nd = dx * cv[sl] + dy * sv[sl]
                        dr2 = dx * dx + dy * dy
                        cone = nd * jnp.abs(nd) > dr2 * _COS2
                        upd = cone & (nd < cnd)
                        cnd = jnp.where(upd, nd, cnd)
                        lx = jnp.where(upd, vxa, lx)
                        ly = jnp.where(upd, vya, ly)
                        nc += [cnd, lx, ly]
                    return tuple(nc)

                res = lax.fori_loop(0, n_veh, cand, (inf16, z16, z16) * ng)
                for g in range(ng):
                    cnd, lx, ly = res[3 * g:3 * g + 3]
                    sl = pl.ds(off + g * _SC_L, _SC_L)
                    dvx = lx - vxv[sl]
                    dvy = ly - vyv[sl]
                    ndv = dvx * cv[sl] + dvy * sv[sl]
                    ss = apv[sl] + bpv[sl] * ndv
                    sal = cnd - L16
                    r = ss / sal
                    afg = afv[sl]
                    ov[sl] = jnp.where(sal == inf16, afg, afg - r * r)
                return 0

            lax.fori_loop(0, _CH, batch_body, 0)
            pltpu.sync_copy(ov, out_h.at[pl.ds(cb, chw)])

    return sck(x, y, c, s, vx, vy, ap, bp, af, l16)


def kernel(state, lengths, v0, s0, dth, amax, b):
    B = state.shape[0]
    st = state.reshape(B, -1, 5)
    n = st.shape[1]
    npad = ((n + _SC_L - 1) // _SC_L) * _SC_L
    pad = ((0, 0), (0, npad - n))
    x = jnp.pad(st[..., 0], pad)
    y = jnp.pad(st[..., 1], pad)
    v = jnp.pad(st[..., 2], pad)
    p = jnp.pad(st[..., 3], pad)
    scal = jnp.concatenate(
        [lengths, v0, s0, dth, amax, b]).astype(jnp.float32)
    c, s, vx, vy, ap, bp, af = _tc_prep(x, y, v, p, scal)
    l16 = jnp.full((_SC_L,), lengths[0], jnp.float32)
    flat = [a.reshape(-1) for a in (x, y, c, s, vx, vy, ap, bp, af)]
    out = _sc_idm_call(*flat, l16, n, npad)
    return out.reshape(B, npad)[:, :n, None]


# speedup vs baseline: 2.7537x; 2.7537x over previous
import functools

import numpy as np
import jax
import jax.numpy as jnp
from jax import lax
from jax.experimental import pallas as pl
from jax.experimental.pallas import tpu as pltpu
from jax.experimental.pallas import tpu_sc as plsc

_COS2 = float(np.cos(np.deg2rad(20.0)) ** 2)
_LANES = 128
_SC_L = 16
_CH = 64


def _tc_body(scal_ref, x_ref, y_ref, v_ref, p_ref, o_ref, vx_ref, vy_ref,
             *, n_veh: int, n_pad: int):
    X = x_ref[...]
    Y = y_ref[...]
    V = v_ref[...]
    P = p_ref[...]
    C = jnp.cos(P)
    S = jnp.sin(P)
    VX = V * C
    VY = V * S
    vx_ref[...] = VX
    vy_ref[...] = VY
    inf = jnp.float32(np.inf)
    cnd0 = jnp.full((n_pad, _LANES), inf, jnp.float32)
    z = jnp.zeros((n_pad, _LANES), jnp.float32)

    def body(a, carry):
        cnd, lvx, lvy = carry
        xa = x_ref[pl.ds(a, 1), :]
        ya = y_ref[pl.ds(a, 1), :]
        vxa = vx_ref[pl.ds(a, 1), :]
        vya = vy_ref[pl.ds(a, 1), :]
        dx = xa - X
        dy = ya - Y
        nd = dx * C + dy * S
        dr2 = dx * dx + dy * dy
        cone = nd * jnp.abs(nd) > dr2 * _COS2
        upd = cone & (nd < cnd)
        cnd = jnp.where(upd, nd, cnd)
        lvx = jnp.where(upd, vxa, lvx)
        lvy = jnp.where(upd, vya, lvy)
        return cnd, lvx, lvy

    cnd, lvx, lvy = lax.fori_loop(0, n_veh, body, (cnd0, z, z))

    L = scal_ref[0]
    v0 = scal_ref[1]
    s0 = scal_ref[2]
    dth = scal_ref[3]
    amax = scal_ref[4]
    bb = scal_ref[5]
    inv2 = 0.5 * lax.rsqrt(amax * bb)
    dvx = lvx - VX
    dvy = lvy - VY
    ndv = dvx * C + dvy * S
    sstar = s0 + V * dth + V * ndv * inv2
    sal = cnd - L
    t = V * (1.0 / v0)
    t2 = t * t
    af = amax * (1.0 - t2 * t2)
    r = sstar / sal
    act = af - amax * (r * r)
    o_ref[...] = jnp.where(sal == inf, af, act)


def _tc_idm(state, scal, n, n_pad):
    B = state.shape[0]
    st = state.reshape(B, -1, 5)
    pad = ((0, n_pad - n), (0, 0))
    xt = jnp.pad(st[..., 0].T, pad)
    yt = jnp.pad(st[..., 1].T, pad)
    vt = jnp.pad(st[..., 2].T, pad)
    pt = jnp.pad(st[..., 3].T, pad)
    grid = (B // _LANES,)
    body = functools.partial(_tc_body, n_veh=n, n_pad=n_pad)
    vspec = pl.BlockSpec((n_pad, _LANES), lambda i: (0, i))
    out = pl.pallas_call(
        body,
        grid=grid,
        in_specs=[
            pl.BlockSpec(memory_space=pltpu.SMEM),
            vspec, vspec, vspec, vspec,
        ],
        out_specs=vspec,
        out_shape=jax.ShapeDtypeStruct((n_pad, B), jnp.float32),
        scratch_shapes=[
            pltpu.VMEM((n_pad, _LANES), jnp.float32),
            pltpu.VMEM((n_pad, _LANES), jnp.float32),
        ],
        compiler_params=pltpu.CompilerParams(
            dimension_semantics=("parallel",)),
    )(scal, xt, yt, vt, pt)
    return out[:n].T


def _prep_body(scal_ref, v_ref, p_ref,
               c_ref, s_ref, vx_ref, vy_ref, ap_ref, bp_ref, af_ref):
    v = v_ref[...]
    p = p_ref[...]
    c = jnp.cos(p)
    s = jnp.sin(p)
    v0 = scal_ref[1]
    s0 = scal_ref[2]
    dth = scal_ref[3]
    amax = scal_ref[4]
    bb = scal_ref[5]
    c_ref[...] = c
    s_ref[...] = s
    vx_ref[...] = v * c
    vy_ref[...] = v * s
    sqa = jnp.sqrt(amax)
    ap_ref[...] = sqa * (s0 + v * dth)
    bp_ref[...] = (0.5 * lax.rsqrt(bb)) * v
    t = v * (1.0 / v0)
    t2 = t * t
    af_ref[...] = amax * (1.0 - t2 * t2)


def _tc_prep(x, y, v, p, scal):
    B, npad = v.shape
    blk = min(512, B)
    vspec = pl.BlockSpec((blk, npad), lambda i: (i, 0))
    shp = jax.ShapeDtypeStruct((B, npad), jnp.float32)
    return pl.pallas_call(
        _prep_body,
        grid=(B // blk,),
        in_specs=[pl.BlockSpec(memory_space=pltpu.SMEM), vspec, vspec],
        out_specs=[vspec] * 7,
        out_shape=[shp] * 7,
        compiler_params=pltpu.CompilerParams(
            dimension_semantics=("parallel",)),
    )(scal, v, p)


def _sc_idm_call(x, y, c, s, vx, vy, ap, bp, af, l16, n_veh, npad):
    B = x.shape[0] // npad
    ng = npad // _SC_L
    n_workers = 32
    per_w = B // n_workers
    ch = min(_CH, per_w)
    chw = ch * npad
    mesh = plsc.VectorSubcoreMesh(core_axis_name="c", subcore_axis_name="s")

    @functools.partial(
        pl.kernel, mesh=mesh,
        out_type=jax.ShapeDtypeStruct((B * npad,), jnp.float32),
        scratch_types=[pltpu.VMEM((chw,), jnp.float32)] * 10
                      + [pltpu.VMEM((_SC_L,), jnp.float32)],
        compiler_params=pltpu.CompilerParams(needs_layout_passes=False),
    )
    def sck(x_h, y_h, c_h, s_h, vx_h, vy_h, ap_h, bp_h, af_h, l_h, out_h,
            xv, yv, cv, sv, vxv, vyv, apv, bpv, afv, ov, lv):
        wid = lax.axis_index("s") * 2 + lax.axis_index("c")
        base0 = wid * (per_w * npad)
        pltpu.sync_copy(l_h, lv)
        L16 = lv[...]
        inf16 = jnp.full((_SC_L,), np.inf, jnp.float32)
        z16 = jnp.zeros((_SC_L,), jnp.float32)
        zi = jnp.zeros((_SC_L,), jnp.int32)

        for ci in range(per_w // ch):
            cb = base0 + ci * chw
            for src, dst in ((x_h, xv), (y_h, yv), (c_h, cv), (s_h, sv),
                             (vx_h, vxv), (vy_h, vyv), (ap_h, apv),
                             (bp_h, bpv), (af_h, afv)):
                pltpu.sync_copy(src.at[pl.ds(cb, chw)], dst)

            def batch_body(bi, _):
                off = bi * npad

                def cand(a, carry):
                    ia = zi + (off + a)
                    xa = plsc.load_gather(xv, [ia])
                    ya = plsc.load_gather(yv, [ia])
                    vxa = plsc.load_gather(vxv, [ia])
                    vya = plsc.load_gather(vyv, [ia])
                    nc = []
                    for g in range(ng):
                        cnd, lx, ly = carry[3 * g:3 * g + 3]
                        sl = pl.ds(off + g * _SC_L, _SC_L)
                        dx = xa - xv[sl]
                        dy = ya - yv[sl]
                        nd = dx * cv[sl] + dy * sv[sl]
                        dr2 = dx * dx + dy * dy
                        cone = nd * jnp.abs(nd) > dr2 * _COS2
                        upd = cone & (nd < cnd)
                        cnd = jnp.where(upd, nd, cnd)
                        lx = jnp.where(upd, vxa, lx)
                        ly = jnp.where(upd, vya, ly)
                        nc += [cnd, lx, ly]
                    return tuple(nc)

                res = lax.fori_loop(0, n_veh, cand, (inf16, z16, z16) * ng)
                for g in range(ng):
                    cnd, lx, ly = res[3 * g:3 * g + 3]
                    sl = pl.ds(off + g * _SC_L, _SC_L)
                    dvx = lx - vxv[sl]
                    dvy = ly - vyv[sl]
                    ndv = dvx * cv[sl] + dvy * sv[sl]
                    ss = apv[sl] + bpv[sl] * ndv
                    sal = cnd - L16
                    r = ss / sal
                    afg = afv[sl]
                    ov[sl] = jnp.where(sal == inf16, afg, afg - r * r)
                return 0

            lax.fori_loop(0, ch, batch_body, 0)
            pltpu.sync_copy(ov, out_h.at[pl.ds(cb, chw)])

    return sck(x, y, c, s, vx, vy, ap, bp, af, l16)


def _sc_idm(state, lengths, scal, n, npad):
    B = state.shape[0]
    st = state.reshape(B, -1, 5)
    pad = ((0, 0), (0, npad - n))
    x = jnp.pad(st[..., 0], pad)
    y = jnp.pad(st[..., 1], pad)
    v = jnp.pad(st[..., 2], pad)
    p = jnp.pad(st[..., 3], pad)
    c, s, vx, vy, ap, bp, af = _tc_prep(x, y, v, p, scal)
    l16 = jnp.full((_SC_L,), lengths[0], jnp.float32)
    flat = [a.reshape(-1) for a in (x, y, c, s, vx, vy, ap, bp, af)]
    out = _sc_idm_call(*flat, l16, n, npad)
    return out.reshape(B, npad)[:, :n]


_TC_SHARE = 0.75


def kernel(state, lengths, v0, s0, dth, amax, b):
    B = state.shape[0]
    st = state.reshape(B, -1, 5)
    n = st.shape[1]
    n_pad8 = ((n + 7) // 8) * 8
    npad = ((n + _SC_L - 1) // _SC_L) * _SC_L
    scal = jnp.concatenate(
        [lengths, v0, s0, dth, amax, b]).astype(jnp.float32)

    b_tc = int(round(B * _TC_SHARE / _LANES)) * _LANES
    b_sc = B - b_tc
    if b_sc % (32 * 8) != 0 or b_sc == 0 or b_tc == 0:
        b_tc, b_sc = B, 0

    out_tc = _tc_idm(state[:b_tc], scal, n, n_pad8)
    if b_sc:
        out_sc = _sc_idm(state[b_tc:], lengths, scal, n, npad)
        out = jnp.concatenate([out_tc, out_sc], axis=0)
    else:
        out = out_tc
    return out[..., None]
